# Initial kernel scaffold; baseline (speedup 1.0000x reference)
import jax
import jax.numpy as jnp
from jax.experimental import pallas as pl


def kernel(pos, atomic_numbers, edge_index, atom_emb, W_rbf, b_rbf, W_src, W_upd):
    raise NotImplementedError("write your pallas kernel here")


# speedup vs baseline: 3.7146x; 3.7146x over previous
import functools

import jax
import jax.numpy as jnp
from jax import lax
from jax.experimental import pallas as pl
from jax.experimental.pallas import tpu as pltpu
from jax.experimental.pallas import tpu_sc as plsc

N = 10000
E = 320000
C = 128
NUM_RBF = 512
MAX_RADIUS = 12.0
NUM_LAYERS = 2

NC = 2
NS = 16
NW = NC * NS
EPW = E // NW

ZQ = 4
ZCH = 80
NPW = ZQ * ZCH
NPAD = NW * NPW

CH = 80
NSTEPS = EPW // CH
STRIPE = 640
LAST_STRIPE = N - (NS - 1) * STRIPE

_sc_params = pltpu.CompilerParams(needs_layout_passes=False)


@functools.cache
def _mesh():
    return plsc.VectorSubcoreMesh(core_axis_name="c", subcore_axis_name="s")


def _geom_body(px_h, py_h, pz_h, jj_h, ii_h, zn_h, emb_h,
               d2_h, x0_h,
               px_v, py_v, pz_v, jv, iv, d2v, znv, rows_v, sem):
    cid = lax.axis_index("c")
    sid = lax.axis_index("s")
    wid = cid * NS + sid
    base = wid * EPW

    pltpu.sync_copy(px_h, px_v)
    pltpu.sync_copy(py_h, py_v)
    pltpu.sync_copy(pz_h, pz_v)
    pltpu.sync_copy(jj_h.at[pl.ds(base, EPW)], jv)
    pltpu.sync_copy(ii_h.at[pl.ds(base, EPW)], iv)

    def body(k, carry):
        s = pl.ds(k * 16, 16)
        jk = jv[s]
        ik = iv[s]
        dx = plsc.load_gather(px_v, [jk]) - plsc.load_gather(px_v, [ik])
        dy = plsc.load_gather(py_v, [jk]) - plsc.load_gather(py_v, [ik])
        dz = plsc.load_gather(pz_v, [jk]) - plsc.load_gather(pz_v, [ik])
        d2v[s] = dx * dx + dy * dy + dz * dz
        return carry

    lax.fori_loop(0, EPW // 16, body, 0)
    pltpu.sync_copy(d2v, d2_h.at[pl.ds(base, EPW)])

    pltpu.sync_copy(zn_h.at[pl.ds(wid * NPW, NPW)], znv)
    for q in range(ZQ):
        pltpu.async_copy(emb_h.at[znv.at[pl.ds(q * ZCH, ZCH)]],
                         rows_v.at[pl.ds(q * ZCH, ZCH)], sem).wait()
    pltpu.sync_copy(rows_v, x0_h.at[pl.ds(wid * NPW, NPW)])


@functools.cache
def _sc_geom():
    return pl.kernel(
        _geom_body,
        out_type=(jax.ShapeDtypeStruct((E,), jnp.float32),
                  jax.ShapeDtypeStruct((NPAD, C), jnp.float32)),
        mesh=_mesh(),
        compiler_params=_sc_params,
        scratch_types=[
            pltpu.VMEM((N,), jnp.float32),
            pltpu.VMEM((N,), jnp.float32),
            pltpu.VMEM((N,), jnp.float32),
            pltpu.VMEM((EPW,), jnp.int32),
            pltpu.VMEM((EPW,), jnp.int32),
            pltpu.VMEM((EPW,), jnp.float32),
            pltpu.VMEM((NPW,), jnp.int32),
            pltpu.VMEM((NPW, C), jnp.float32),
            pltpu.SemaphoreType.DMA,
        ],
    )


def _msg_body(h_h, e_h, jj_h, ii_h, zer_h, out_h,
              agg_s, jv, iv, hbuf, ebuf, sem, gsem):
    cid = lax.axis_index("c")
    sid = lax.axis_index("s")
    wid = cid * NS + sid

    @pl.when(sid == 0)
    def _():
        pltpu.sync_copy(zer_h, agg_s)

    plsc.subcore_barrier()

    def step(t, carry):
        eb = wid * EPW + t * CH
        pltpu.sync_copy(jj_h.at[pl.ds(eb, CH)], jv)
        pltpu.sync_copy(ii_h.at[pl.ds(eb, CH)], iv)
        pltpu.async_copy(h_h.at[jv], hbuf, gsem).wait()
        pltpu.sync_copy(e_h.at[pl.ds(eb, CH)], ebuf)

        def row(r, c2):
            for g in range(8):
                cs = pl.ds(g * 16, 16)
                hbuf[r, cs] = hbuf[r, cs] * ebuf[r, cs]
            return c2

        lax.fori_loop(0, CH, row, 0)
        pltpu.sync_copy(hbuf, agg_s.at[iv], add=True)
        return carry

    lax.fori_loop(0, NSTEPS, step, 0)
    plsc.subcore_barrier()
    rb = sid * STRIPE

    @pl.when(sid < NS - 1)
    def _():
        pltpu.sync_copy(agg_s.at[pl.ds(rb, STRIPE)],
                        out_h.at[cid, pl.ds(rb, STRIPE)])

    @pl.when(sid == NS - 1)
    def _():
        pltpu.sync_copy(agg_s.at[pl.ds((NS - 1) * STRIPE, LAST_STRIPE)],
                        out_h.at[cid, pl.ds((NS - 1) * STRIPE, LAST_STRIPE)])


@functools.cache
def _sc_msg():
    return pl.kernel(
        _msg_body,
        out_type=jax.ShapeDtypeStruct((NC, N, C), jnp.float32),
        mesh=_mesh(),
        compiler_params=_sc_params,
        scratch_types=[
            pltpu.VMEM_SHARED((N, C), jnp.float32),
            pltpu.VMEM((CH,), jnp.int32),
            pltpu.VMEM((CH,), jnp.int32),
            pltpu.VMEM((CH, C), jnp.float32),
            pltpu.VMEM((CH, C), jnp.float32),
            pltpu.SemaphoreType.DMA,
            pltpu.SemaphoreType.DMA,
        ],
    )


TE = 2000
NB = E // TE
SIGMA = MAX_RADIUS / NUM_RBF
DC = MAX_RADIUS / (NUM_RBF - 1)


def _rbf_body(d2_ref, w_ref, b_ref, o_ref):
    d2 = d2_ref[0, 0, :]
    dist = jnp.sqrt(d2 + 1e-12)
    kk = lax.broadcasted_iota(jnp.int32, (TE, NUM_RBF), 1).astype(jnp.float32)
    zz = (dist[:, None] - kk * DC) * (1.0 / SIGMA)
    rbf = jnp.exp(-0.5 * zz * zz)
    y = jnp.dot(rbf, w_ref[...], preferred_element_type=jnp.float32)
    y = y + b_ref[...]
    o_ref[...] = y * jax.nn.sigmoid(y)


def _tc_rbf(d2, w, b):
    d2r = d2.reshape(NB, 1, TE)
    return pl.pallas_call(
        _rbf_body,
        grid=(NB,),
        in_specs=[
            pl.BlockSpec((1, 1, TE), lambda t: (t, 0, 0)),
            pl.BlockSpec((NUM_RBF, C), lambda t: (0, 0)),
            pl.BlockSpec((1, C), lambda t: (0, 0)),
        ],
        out_specs=pl.BlockSpec((TE, C), lambda t: (t, 0)),
        out_shape=jax.ShapeDtypeStruct((E, C), jnp.float32),
    )(d2r, w, b.reshape(1, C))


TN = 2000
NNB = N // TN


def _h_body(x_ref, w_ref, o_ref):
    y = jnp.dot(x_ref[...], w_ref[...], preferred_element_type=jnp.float32)
    o_ref[...] = y * jax.nn.sigmoid(y)


def _tc_h(x, w):
    return pl.pallas_call(
        _h_body,
        grid=(NNB,),
        in_specs=[
            pl.BlockSpec((TN, C), lambda t: (t, 0)),
            pl.BlockSpec((C, C), lambda t: (0, 0)),
        ],
        out_specs=pl.BlockSpec((TN, C), lambda t: (t, 0)),
        out_shape=jax.ShapeDtypeStruct((N, C), jnp.float32),
    )(x, w)


def _upd_body(x_ref, p_ref, w_ref, o_ref):
    agg = p_ref[0] + p_ref[1]
    u = jnp.dot(agg, w_ref[...], preferred_element_type=jnp.float32)
    u = u * jax.nn.sigmoid(u)
    y = x_ref[...] + u
    ms = jnp.mean(y * y, axis=-1, keepdims=True)
    o_ref[...] = y * lax.rsqrt(ms + 1e-6)


def _tc_upd(x, parts, w):
    return pl.pallas_call(
        _upd_body,
        grid=(NNB,),
        in_specs=[
            pl.BlockSpec((TN, C), lambda t: (t, 0)),
            pl.BlockSpec((NC, TN, C), lambda t: (0, t, 0)),
            pl.BlockSpec((C, C), lambda t: (0, 0)),
        ],
        out_specs=pl.BlockSpec((TN, C), lambda t: (t, 0)),
        out_shape=jax.ShapeDtypeStruct((N, C), jnp.float32),
    )(x, parts, w)


def kernel(pos, atomic_numbers, edge_index, atom_emb, W_rbf, b_rbf, W_src, W_upd):
    pos = pos.astype(jnp.float32)
    j = edge_index[0].astype(jnp.int32)
    i = edge_index[1].astype(jnp.int32)
    px = pos[:, 0]
    py = pos[:, 1]
    pz = pos[:, 2]
    zn = jnp.concatenate(
        [atomic_numbers.astype(jnp.int32),
         jnp.zeros((NPAD - N,), jnp.int32)])
    zer = jnp.zeros((N, C), jnp.float32)

    d2, x0 = _sc_geom()(px, py, pz, j, i, zn, atom_emb)
    x = x0[:N]
    e = _tc_rbf(d2, W_rbf, b_rbf)
    for l in range(NUM_LAYERS):
        h = _tc_h(x, W_src[l])
        parts = _sc_msg()(h, e, j, i, zer)
        x = _tc_upd(x, parts, W_upd[l])
    return x
